# baseline (speedup 1.0000x reference)
import functools

import jax
import jax.numpy as jnp
from jax import lax
from jax.experimental import pallas as pl
from jax.experimental.pallas import tpu as pltpu
from jax.experimental.pallas import tpu_sc as plsc

_N = 1000
_D = 4096
_TOT = _N * _D
_NW = 32
_PER_W = _TOT // _NW

_mesh = plsc.VectorSubcoreMesh(core_axis_name="c", subcore_axis_name="s")


@functools.partial(
    pl.kernel,
    mesh=_mesh,
    out_type=jax.ShapeDtypeStruct((_TOT,), jnp.float32),
)
def _copy_kernel(src_hbm, out_hbm):
    wid = lax.axis_index("s") * 2 + lax.axis_index("c")
    base = wid * _PER_W
    pltpu.sync_copy(src_hbm.at[pl.ds(base, _PER_W)],
                    out_hbm.at[pl.ds(base, _PER_W)])


def kernel(special_embeddings_weight):
    flat = special_embeddings_weight.reshape(_TOT)
    return _copy_kernel(flat).reshape(_N, _D)


# speedup vs baseline: 8.5409x; 8.5409x over previous
import functools

import jax
import jax.numpy as jnp
from jax import lax
from jax.experimental import pallas as pl
from jax.experimental.pallas import tpu as pltpu
from jax.experimental.pallas import tpu_sc as plsc

_N = 1000
_D = 4096
_TOT = _N * _D
_NW = 32
_PER_W = _TOT // _NW

_mesh = plsc.VectorSubcoreMesh(core_axis_name="c", subcore_axis_name="s")


@functools.partial(
    pl.kernel,
    mesh=_mesh,
    out_type=jax.ShapeDtypeStruct((_TOT,), jnp.float32),
    scratch_types=[pltpu.VMEM((_PER_W,), jnp.float32)],
)
def _copy_kernel(src_hbm, out_hbm, buf):
    wid = lax.axis_index("s") * 2 + lax.axis_index("c")
    base = wid * _PER_W
    pltpu.sync_copy(src_hbm.at[pl.ds(base, _PER_W)], buf)
    pltpu.sync_copy(buf, out_hbm.at[pl.ds(base, _PER_W)])


def kernel(special_embeddings_weight):
    flat = special_embeddings_weight.reshape(_TOT)
    return _copy_kernel(flat).reshape(_N, _D)


# speedup vs baseline: 16.7608x; 1.9624x over previous
import functools

import jax
import jax.numpy as jnp
from jax import lax
from jax.experimental import pallas as pl
from jax.experimental.pallas import tpu as pltpu
from jax.experimental.pallas import tpu_sc as plsc

_N = 1000
_D = 4096
_CHUNK_ROWS = 8
_NCHUNKS = _N // _CHUNK_ROWS
_NW = 32
_ITERS = -(-_NCHUNKS // _NW)

_mesh = plsc.VectorSubcoreMesh(core_axis_name="c", subcore_axis_name="s")


@functools.partial(
    pl.kernel,
    mesh=_mesh,
    out_type=jax.ShapeDtypeStruct((_N, _D), jnp.float32),
    scratch_types=[pltpu.VMEM((_CHUNK_ROWS, _D), jnp.float32)],
)
def _copy_kernel(src_hbm, out_hbm, buf):
    wid = lax.axis_index("s") * 2 + lax.axis_index("c")

    def copy_chunk(c):
        r = c * _CHUNK_ROWS
        pltpu.sync_copy(src_hbm.at[pl.ds(r, _CHUNK_ROWS), :], buf)
        pltpu.sync_copy(buf, out_hbm.at[pl.ds(r, _CHUNK_ROWS), :])

    for i in range(_ITERS):
        c = wid + _NW * i
        if (i + 1) * _NW <= _NCHUNKS:
            copy_chunk(c)
        else:
            @pl.when(c < _NCHUNKS)
            def _():
                copy_chunk(c)


def kernel(special_embeddings_weight):
    return _copy_kernel(special_embeddings_weight)


# speedup vs baseline: 17.6426x; 1.0526x over previous
import functools

import jax
import jax.numpy as jnp
from jax import lax
from jax.experimental import pallas as pl
from jax.experimental.pallas import tpu as pltpu
from jax.experimental.pallas import tpu_sc as plsc

_N = 1000
_D = 4096
_CHUNK_ROWS = 8
_NCHUNKS = _N // _CHUNK_ROWS
_NW = 32
_ITERS = -(-_NCHUNKS // _NW)

_mesh = plsc.VectorSubcoreMesh(core_axis_name="c", subcore_axis_name="s")


@functools.partial(
    pl.kernel,
    mesh=_mesh,
    out_type=jax.ShapeDtypeStruct((_N, _D), jnp.float32),
    scratch_types=[
        pltpu.VMEM((_CHUNK_ROWS, _D), jnp.float32),
        pltpu.VMEM((_CHUNK_ROWS, _D), jnp.float32),
        pltpu.VMEM((_CHUNK_ROWS, _D), jnp.float32),
        pltpu.SemaphoreType.DMA,
        pltpu.SemaphoreType.DMA,
        pltpu.SemaphoreType.DMA,
        pltpu.SemaphoreType.DMA,
        pltpu.SemaphoreType.DMA,
        pltpu.SemaphoreType.DMA,
    ],
)
def _copy_kernel(src_hbm, out_hbm, b0, b1, b2, si0, si1, si2, so0, so1, so2):
    wid = lax.axis_index("s") * 2 + lax.axis_index("c")
    bufs = (b0, b1, b2)
    sins = (si0, si1, si2)
    souts = (so0, so1, so2)

    def start_in(i):
        r = (wid + _NW * i) * _CHUNK_ROWS
        return pltpu.async_copy(
            src_hbm.at[pl.ds(r, _CHUNK_ROWS), :], bufs[i % 3], sins[i % 3])

    def start_out(i):
        r = (wid + _NW * i) * _CHUNK_ROWS
        return pltpu.async_copy(
            bufs[i % 3], out_hbm.at[pl.ds(r, _CHUNK_ROWS), :], souts[i % 3])

    h_in0 = start_in(0)
    h_in1 = start_in(1)
    h_in2 = start_in(2)
    h_in0.wait()
    h_out0 = start_out(0)
    h_in1.wait()
    h_out1 = start_out(1)
    h_in2.wait()
    h_out2 = start_out(2)
    h_out0.wait()

    @pl.when(wid + _NW * 3 < _NCHUNKS)
    def _():
        h_in3 = start_in(3)
        h_in3.wait()
        h_out3 = start_out(3)
        h_out3.wait()

    h_out1.wait()
    h_out2.wait()


def kernel(special_embeddings_weight):
    return _copy_kernel(special_embeddings_weight)
